# baseline (speedup 1.0000x reference)
import functools

import jax
import jax.numpy as jnp
from jax import lax
from jax.experimental import pallas as pl
from jax.experimental.pallas import tpu as pltpu
from jax.experimental.pallas import tpu_sc as plsc

K = 1024
D = 64
BETA = 0.25
N_BATCH = 16
T = 1024
N_TOK = N_BATCH * T


def _dist_kernel(x_ref, e_ref, idx_ref, loss_ref, en_ref, e2_ref):
    ze = x_ref[0].reshape(D, T)
    e = e_ref[...]

    @pl.when(pl.program_id(0) == 0)
    def _():
        en_ref[...] = jnp.sum(e * e, axis=1, keepdims=True)
        e2_ref[...] = -2.0 * e

    zn = jnp.sum(ze * ze, axis=0, keepdims=True)
    s2 = lax.dot_general(e2_ref[...], ze, (((1,), (0,)), ((), ())),
                         preferred_element_type=jnp.float32)
    dist = (zn + en_ref[...]) + s2
    minv = jnp.min(dist, axis=0, keepdims=True)
    col = lax.broadcasted_iota(jnp.int32, (K, 1), 0).astype(jnp.float32)
    idxf = jnp.min(jnp.where(dist == minv, col, float(K)), axis=0,
                   keepdims=True)
    idx_ref[...] = idxf.astype(jnp.int32).reshape(8, 128)
    part = jnp.sum(minv, keepdims=True)

    @pl.when(pl.program_id(0) == 0)
    def _():
        loss_ref[...] = part

    @pl.when(pl.program_id(0) != 0)
    def _():
        loss_ref[...] += part


def _distances(x, e_weight):
    return pl.pallas_call(
        _dist_kernel,
        grid=(N_BATCH,),
        in_specs=[
            pl.BlockSpec((1, D, 32, 32), lambda n: (n, 0, 0, 0)),
            pl.BlockSpec((K, D), lambda n: (0, 0)),
        ],
        out_specs=[
            pl.BlockSpec((8, 128), lambda n: (n, 0)),
            pl.BlockSpec((1, 1), lambda n: (0, 0)),
        ],
        out_shape=[
            jax.ShapeDtypeStruct((128, 128), jnp.int32),
            jax.ShapeDtypeStruct((1, 1), jnp.float32),
        ],
        scratch_shapes=[pltpu.VMEM((K, 1), jnp.float32),
                        pltpu.VMEM((K, D), jnp.float32)],
    )(x, e_weight)


_NC, _NS = 2, 16
_NW = _NC * _NS
_TPW = N_TOK // _NW
_CHUNK = 128
_NCHUNK = _TPW // _CHUNK


def _sc_gather_kernel(idx_hbm, table_hbm, out_hbm, hist_hbm,
                      idx_v, rows_v, hist_v, sem):
    wid = lax.axis_index("s") * _NC + lax.axis_index("c")
    base = wid * _NCHUNK
    pltpu.sync_copy(idx_hbm.at[pl.ds(base, _NCHUNK)], idx_v)
    handles = [
        pltpu.async_copy(table_hbm.at[idx_v.at[j]], rows_v.at[j], sem)
        for j in range(_NCHUNK)
    ]
    zeros = jnp.zeros((16,), jnp.float32)
    for i in range(K // 16):
        hist_v[pl.ds(i * 16, 16)] = zeros
    ones = jnp.ones((16,), jnp.float32)
    for j in range(_NCHUNK):
        for t in range(_CHUNK // 16):
            v = idx_v[j, pl.ds(t * 16, 16)]
            plsc.addupdate_scatter(hist_v, [v], ones)
    for h in handles:
        h.wait()
    pltpu.sync_copy(rows_v, out_hbm.at[pl.ds(base, _NCHUNK)])
    pltpu.sync_copy(hist_v, hist_hbm.at[wid])


@functools.cache
def _sc_gather():
    return pl.kernel(
        _sc_gather_kernel,
        mesh=plsc.VectorSubcoreMesh(core_axis_name="c", subcore_axis_name="s"),
        out_type=[
            jax.ShapeDtypeStruct((_NW * _NCHUNK, _CHUNK, D), jnp.float32),
            jax.ShapeDtypeStruct((_NW, K), jnp.float32),
        ],
        scratch_types=[
            pltpu.VMEM((_NCHUNK, _CHUNK), jnp.int32),
            pltpu.VMEM((_NCHUNK, _CHUNK, D), jnp.float32),
            pltpu.VMEM((K,), jnp.float32),
            pltpu.SemaphoreType.DMA,
        ],
        compiler_params=pltpu.CompilerParams(
            needs_layout_passes=False, use_tc_tiling_on_sc=False),
    )


def _finalize_kernel(hist_ref, loss_ref, eq_ref, el_ref, ql_ref, est_ref):
    h = hist_ref[...]
    counts = jnp.sum(h, axis=0, keepdims=True)
    tot = jnp.sum(counts)
    prob = counts / tot
    log_prob = jnp.log2(prob + 1e-10)
    ent = -jnp.sum(prob * log_prob, keepdims=True)
    est_ref[...] = jnp.exp(ent * 0.6931471805599453)
    el = loss_ref[...] * (1.0 / (N_TOK * D))
    el_ref[...] = el
    ql_ref[...] = el
    eq_ref[...] = el + BETA * el


def _finalize(hist, loss_sum):
    return pl.pallas_call(
        _finalize_kernel,
        in_specs=[
            pl.BlockSpec((_NW, K), lambda: (0, 0)),
            pl.BlockSpec((1, 1), lambda: (0, 0)),
        ],
        out_specs=[pl.BlockSpec((1, 1), lambda: (0, 0))] * 4,
        out_shape=[jax.ShapeDtypeStruct((1, 1), jnp.float32)] * 4,
    )(hist, loss_sum)


def kernel(inputs, E_weight):
    idx128, loss_sum = _distances(inputs, E_weight)
    zq_rows, hist = _sc_gather()(idx128, E_weight)
    eq, el, ql, est = _finalize(hist, loss_sum)
    zq = zq_rows.reshape(N_BATCH, 32, 32, D).transpose(0, 3, 1, 2)
    return (eq[0, 0], zq, el[0, 0], ql[0, 0], est[0, 0])


# speedup vs baseline: 1.2076x; 1.2076x over previous
import functools

import jax
import jax.numpy as jnp
from jax import lax
from jax.experimental import pallas as pl
from jax.experimental.pallas import tpu as pltpu
from jax.experimental.pallas import tpu_sc as plsc

K = 1024
D = 64
BETA = 0.25
N_BATCH = 16
T = 1024
N_TOK = N_BATCH * T


def _dist_kernel(x_ref, e_ref, idx_ref, loss_ref, en_ref, e2_ref):
    ze = x_ref[0]
    e = e_ref[...]

    @pl.when(pl.program_id(0) == 0)
    def _():
        en_ref[...] = jnp.sum(e * e, axis=1, keepdims=True)
        e2_ref[...] = -2.0 * e

    zn = jnp.sum(ze * ze, axis=0, keepdims=True)
    s2 = lax.dot_general(e2_ref[...], ze, (((1,), (0,)), ((), ())),
                         preferred_element_type=jnp.float32)
    dist = (zn + en_ref[...]) + s2
    minv = jnp.min(dist, axis=0, keepdims=True)
    col = lax.broadcasted_iota(jnp.int32, (K, 1), 0).astype(jnp.float32)
    idxf = jnp.min(jnp.where(dist == minv, col, float(K)), axis=0,
                   keepdims=True)
    idx_ref[0] = idxf.astype(jnp.int32)
    part = jnp.sum(minv, keepdims=True)

    @pl.when(pl.program_id(0) == 0)
    def _():
        loss_ref[...] = part

    @pl.when(pl.program_id(0) != 0)
    def _():
        loss_ref[...] += part


def _distances(x, e_weight):
    return pl.pallas_call(
        _dist_kernel,
        grid=(N_BATCH,),
        in_specs=[
            pl.BlockSpec((1, D, T), lambda n: (n, 0, 0)),
            pl.BlockSpec((K, D), lambda n: (0, 0)),
        ],
        out_specs=[
            pl.BlockSpec((1, 1, T), lambda n: (n, 0, 0)),
            pl.BlockSpec((1, 1), lambda n: (0, 0)),
        ],
        out_shape=[
            jax.ShapeDtypeStruct((N_BATCH, 1, T), jnp.int32),
            jax.ShapeDtypeStruct((1, 1), jnp.float32),
        ],
        scratch_shapes=[pltpu.VMEM((K, 1), jnp.float32),
                        pltpu.VMEM((K, D), jnp.float32)],
    )(x, e_weight)


_NC, _NS = 2, 16
_NW = _NC * _NS
_TPW = N_TOK // _NW
_CHUNK = 128
_NCHUNK = _TPW // _CHUNK


def _sc_gather_kernel(idx_hbm, table_hbm, out_hbm, hist_hbm,
                      idx_v, rows_v, hist_v, sem):
    wid = lax.axis_index("s") * _NC + lax.axis_index("c")
    base = wid * _NCHUNK
    pltpu.sync_copy(idx_hbm.at[pl.ds(base, _NCHUNK)], idx_v)
    handles = [
        pltpu.async_copy(table_hbm.at[idx_v.at[j]], rows_v.at[j], sem)
        for j in range(_NCHUNK)
    ]
    zeros = jnp.zeros((16,), jnp.float32)
    for i in range(K // 16):
        hist_v[pl.ds(i * 16, 16)] = zeros
    ones = jnp.ones((16,), jnp.float32)
    for j in range(_NCHUNK):
        for t in range(_CHUNK // 16):
            v = idx_v[j, pl.ds(t * 16, 16)]
            plsc.addupdate_scatter(hist_v, [v], ones)
    for h in handles:
        h.wait()
    pltpu.sync_copy(rows_v, out_hbm.at[pl.ds(base, _NCHUNK)])
    pltpu.sync_copy(hist_v, hist_hbm.at[wid])


@functools.cache
def _sc_gather():
    return pl.kernel(
        _sc_gather_kernel,
        mesh=plsc.VectorSubcoreMesh(core_axis_name="c", subcore_axis_name="s"),
        out_type=[
            jax.ShapeDtypeStruct((_NW * _NCHUNK, _CHUNK, D), jnp.float32),
            jax.ShapeDtypeStruct((_NW, K), jnp.float32),
        ],
        scratch_types=[
            pltpu.VMEM((_NCHUNK, _CHUNK), jnp.int32),
            pltpu.VMEM((_NCHUNK, _CHUNK, D), jnp.float32),
            pltpu.VMEM((K,), jnp.float32),
            pltpu.SemaphoreType.DMA,
        ],
        compiler_params=pltpu.CompilerParams(
            needs_layout_passes=False, use_tc_tiling_on_sc=False),
    )


def _finalize_kernel(hist_ref, loss_ref, eq_ref, el_ref, ql_ref, est_ref):
    h = hist_ref[...]
    counts = jnp.sum(h, axis=0, keepdims=True)
    tot = jnp.sum(counts)
    prob = counts / tot
    log_prob = jnp.log2(prob + 1e-10)
    ent = -jnp.sum(prob * log_prob, keepdims=True)
    est_ref[...] = jnp.exp(ent * 0.6931471805599453)
    el = loss_ref[...] * (1.0 / (N_TOK * D))
    el_ref[...] = el
    ql_ref[...] = el
    eq_ref[...] = el + BETA * el


def _finalize(hist, loss_sum):
    return pl.pallas_call(
        _finalize_kernel,
        in_specs=[
            pl.BlockSpec((_NW, K), lambda: (0, 0)),
            pl.BlockSpec((1, 1), lambda: (0, 0)),
        ],
        out_specs=[pl.BlockSpec((1, 1), lambda: (0, 0))] * 4,
        out_shape=[jax.ShapeDtypeStruct((1, 1), jnp.float32)] * 4,
    )(hist, loss_sum)


def kernel(inputs, E_weight):
    x = inputs.reshape(N_BATCH, D, T)
    idx3, loss_sum = _distances(x, E_weight)
    idx128 = idx3.reshape(_NW * _NCHUNK, _CHUNK)
    zq_rows, hist = _sc_gather()(idx128, E_weight)
    eq, el, ql, est = _finalize(hist, loss_sum)
    zq = zq_rows.reshape(N_BATCH, 32, 32, D).transpose(0, 3, 1, 2)
    return (eq[0, 0], zq, el[0, 0], ql[0, 0], est[0, 0])


# speedup vs baseline: 1.2298x; 1.0184x over previous
import functools

import jax
import jax.numpy as jnp
from jax import lax
from jax.experimental import pallas as pl
from jax.experimental.pallas import tpu as pltpu
from jax.experimental.pallas import tpu_sc as plsc

K = 1024
D = 64
BETA = 0.25
N_BATCH = 16
T = 1024
N_TOK = N_BATCH * T


def _dist_kernel(x_ref, e_ref, zq_ref, idx_ref, loss_ref, en_ref, e2_ref):
    ze = x_ref[0]
    e = e_ref[...]

    @pl.when(pl.program_id(0) == 0)
    def _():
        en_ref[...] = jnp.sum(e * e, axis=1, keepdims=True)
        e2_ref[...] = -2.0 * e

    zn = jnp.sum(ze * ze, axis=0, keepdims=True)
    s2 = lax.dot_general(e2_ref[...], ze, (((1,), (0,)), ((), ())),
                         preferred_element_type=jnp.float32)
    dist = (zn + en_ref[...]) + s2
    minv = jnp.min(dist, axis=0, keepdims=True)
    col = lax.broadcasted_iota(jnp.int32, (K, 1), 0).astype(jnp.float32)
    idxf = jnp.min(jnp.where(dist == minv, col, float(K)), axis=0,
                   keepdims=True)
    idx_ref[0] = idxf.astype(jnp.int32)
    onehot = (col == idxf).astype(jnp.float32)
    zq_ref[0] = lax.dot_general(e, onehot, (((0,), (0,)), ((), ())),
                                preferred_element_type=jnp.float32)
    part = jnp.sum(minv, keepdims=True)

    @pl.when(pl.program_id(0) == 0)
    def _():
        loss_ref[...] = part

    @pl.when(pl.program_id(0) != 0)
    def _():
        loss_ref[...] += part


def _distances(x, e_weight):
    return pl.pallas_call(
        _dist_kernel,
        grid=(N_BATCH,),
        in_specs=[
            pl.BlockSpec((1, D, T), lambda n: (n, 0, 0)),
            pl.BlockSpec((K, D), lambda n: (0, 0)),
        ],
        out_specs=[
            pl.BlockSpec((1, D, T), lambda n: (n, 0, 0)),
            pl.BlockSpec((1, 1, T), lambda n: (n, 0, 0)),
            pl.BlockSpec((1, 1), lambda n: (0, 0)),
        ],
        out_shape=[
            jax.ShapeDtypeStruct((N_BATCH, D, T), jnp.float32),
            jax.ShapeDtypeStruct((N_BATCH, 1, T), jnp.int32),
            jax.ShapeDtypeStruct((1, 1), jnp.float32),
        ],
        scratch_shapes=[pltpu.VMEM((K, 1), jnp.float32),
                        pltpu.VMEM((K, D), jnp.float32)],
    )(x, e_weight)


_NC, _NS = 2, 16
_NW = _NC * _NS
_TPW = N_TOK // _NW
_CHUNK = 128
_NCHUNK = _TPW // _CHUNK


def _sc_hist_kernel(idx_hbm, hist_hbm, idx_v, hist_v):
    wid = lax.axis_index("s") * _NC + lax.axis_index("c")
    base = wid * _NCHUNK
    pltpu.sync_copy(idx_hbm.at[pl.ds(base, _NCHUNK)], idx_v)
    zeros = jnp.zeros((16,), jnp.float32)
    for i in range(K // 16):
        hist_v[pl.ds(i * 16, 16)] = zeros
    ones = jnp.ones((16,), jnp.float32)
    for j in range(_NCHUNK):
        for t in range(_CHUNK // 16):
            v = idx_v[j, pl.ds(t * 16, 16)]
            plsc.addupdate_scatter(hist_v, [v], ones)
    pltpu.sync_copy(hist_v, hist_hbm.at[wid])


@functools.cache
def _sc_hist():
    return pl.kernel(
        _sc_hist_kernel,
        mesh=plsc.VectorSubcoreMesh(core_axis_name="c", subcore_axis_name="s"),
        out_type=[
            jax.ShapeDtypeStruct((_NW, K), jnp.float32),
        ],
        scratch_types=[
            pltpu.VMEM((_NCHUNK, _CHUNK), jnp.int32),
            pltpu.VMEM((K,), jnp.float32),
        ],
        compiler_params=pltpu.CompilerParams(
            needs_layout_passes=False, use_tc_tiling_on_sc=False),
    )


def _finalize_kernel(hist_ref, loss_ref, eq_ref, el_ref, ql_ref, est_ref):
    h = hist_ref[...]
    counts = jnp.sum(h, axis=0, keepdims=True)
    tot = jnp.sum(counts)
    prob = counts / tot
    log_prob = jnp.log2(prob + 1e-10)
    ent = -jnp.sum(prob * log_prob, keepdims=True)
    est_ref[...] = jnp.exp(ent * 0.6931471805599453)
    el = loss_ref[...] * (1.0 / (N_TOK * D))
    el_ref[...] = el
    ql_ref[...] = el
    eq_ref[...] = el + BETA * el


def _finalize(hist, loss_sum):
    return pl.pallas_call(
        _finalize_kernel,
        in_specs=[
            pl.BlockSpec((_NW, K), lambda: (0, 0)),
            pl.BlockSpec((1, 1), lambda: (0, 0)),
        ],
        out_specs=[pl.BlockSpec((1, 1), lambda: (0, 0))] * 4,
        out_shape=[jax.ShapeDtypeStruct((1, 1), jnp.float32)] * 4,
    )(hist, loss_sum)


def kernel(inputs, E_weight):
    x = inputs.reshape(N_BATCH, D, T)
    zq3, idx3, loss_sum = _distances(x, E_weight)
    idx128 = idx3.reshape(_NW * _NCHUNK, _CHUNK)
    (hist,) = _sc_hist()(idx128)
    eq, el, ql, est = _finalize(hist, loss_sum)
    zq = zq3.reshape(N_BATCH, D, 32, 32)
    return (eq[0, 0], zq, el[0, 0], ql[0, 0], est[0, 0])
